# baseline (speedup 1.0000x reference)
import functools

import jax
import jax.numpy as jnp
from jax import lax
from jax.experimental import pallas as pl
from jax.experimental.pallas import tpu as pltpu
from jax.experimental.pallas import tpu_sc as plsc

VOCAB = 32000
PAD = 0
SMOOTH = 0.1
CONF = 1.0 - SMOOTH
LOW = SMOOTH / (VOCAB - 2)

ROWS = 256
CHUNK = 6400
NCHUNK = VOCAB // CHUNK

N_TOKENS = 8192
SC_CORES = 2
SC_SUBCORES = 16
SC_WORKERS = SC_CORES * SC_SUBCORES
PER_TILE = N_TOKENS // SC_WORKERS
LANES = 16
IDX_W = 128
IDX_ROWS = PER_TILE // IDX_W


@functools.partial(
    pl.kernel,
    mesh=plsc.VectorSubcoreMesh(core_axis_name="c", subcore_axis_name="s"),
    out_type=jax.ShapeDtypeStruct((N_TOKENS,), jnp.float32),
    scratch_types=[
        pltpu.VMEM((PER_TILE,), jnp.int32),
        pltpu.VMEM((IDX_ROWS, IDX_W), jnp.int32),
        pltpu.VMEM((IDX_ROWS, IDX_W), jnp.float32),
        pltpu.SemaphoreType.DMA,
    ],
)
def _sc_gather_pt(tgt_hbm, pred_hbm, out_hbm, tgt_v, idx_v, val_v, sem):
    wid = lax.axis_index("s") * SC_CORES + lax.axis_index("c")
    base = wid * PER_TILE
    pltpu.sync_copy(tgt_hbm.at[pl.ds(base, PER_TILE)], tgt_v)
    for k in range(PER_TILE // LANES):
        q, off = divmod(k * LANES, IDX_W)
        rows = (base + k * LANES) + lax.iota(jnp.int32, LANES)
        idx_v[q, pl.ds(off, LANES)] = rows * VOCAB + tgt_v[pl.ds(k * LANES, LANES)]
    for q in range(IDX_ROWS):
        pltpu.async_copy(pred_hbm.at[idx_v.at[q]], val_v.at[q], sem).wait()
        pltpu.sync_copy(val_v.at[q], out_hbm.at[pl.ds(base + q * IDX_W, IDX_W)])


def _tc_body(pred_ref, tgt_ref, pt_in_ref, out_ref, m_ref, se_ref, sp_ref,
             p0_ref, num_ref, den_ref):
    i = pl.program_id(0)
    j = pl.program_id(1)

    chunk = pred_ref[...]

    @pl.when(j == 0)
    def _init_row_block():
        m_ref[...] = jnp.full((ROWS, 1), -jnp.inf, jnp.float32)
        se_ref[...] = jnp.zeros((ROWS, 1), jnp.float32)
        sp_ref[...] = jnp.zeros((ROWS, 1), jnp.float32)
        p0_ref[...] = chunk[:, 0:1]

    @pl.when((i == 0) & (j == 0))
    def _init_accum():
        num_ref[0, 0] = 0.0
        den_ref[0, 0] = 0.0

    m_old = m_ref[...]
    m_new = jnp.maximum(m_old, jnp.max(chunk, axis=1, keepdims=True))
    alpha = jnp.exp(m_old - m_new)
    cse = jnp.sum(jnp.exp(chunk - m_new), axis=1, keepdims=True)
    se_ref[...] = se_ref[...] * alpha + cse
    sp_ref[...] = sp_ref[...] + jnp.sum(chunk, axis=1, keepdims=True)
    m_ref[...] = m_new

    @pl.when(j == NCHUNK - 1)
    def _finish_row_block():
        tgt = tgt_ref[...]
        pt = pt_in_ref[...]
        L = m_ref[...] + jnp.log(se_ref[...])
        loss = (L - LOW * (sp_ref[...] - p0_ref[...] - pt) - CONF * pt)
        maskf = (tgt != PAD).astype(jnp.float32)
        num_ref[0, 0] += jnp.sum(loss * maskf)
        den_ref[0, 0] += jnp.sum(maskf)

    @pl.when((i == pl.num_programs(0) - 1) & (j == NCHUNK - 1))
    def _emit():
        out_ref[...] = jnp.full(
            (1, 1), num_ref[0, 0] / jnp.maximum(den_ref[0, 0], 1.0),
            jnp.float32)


def kernel(pred, target):
    n = pred.shape[0]
    pt = _sc_gather_pt(target, pred.reshape(-1))
    tgt2d = target.reshape(n, 1)
    out = pl.pallas_call(
        _tc_body,
        grid=(n // ROWS, NCHUNK),
        in_specs=[
            pl.BlockSpec((ROWS, CHUNK), lambda i, j: (i, j)),
            pl.BlockSpec((ROWS, 1), lambda i, j: (i, 0)),
            pl.BlockSpec((ROWS, 1), lambda i, j: (i, 0)),
        ],
        out_specs=pl.BlockSpec((1, 1), lambda i, j: (0, 0)),
        out_shape=jax.ShapeDtypeStruct((1, 1), jnp.float32),
        scratch_shapes=[
            pltpu.VMEM((ROWS, 1), jnp.float32),
            pltpu.VMEM((ROWS, 1), jnp.float32),
            pltpu.VMEM((ROWS, 1), jnp.float32),
            pltpu.VMEM((ROWS, 1), jnp.float32),
            pltpu.SMEM((1, 1), jnp.float32),
            pltpu.SMEM((1, 1), jnp.float32),
        ],
    )(pred, tgt2d, pt.reshape(n, 1))
    return out[0, 0]


# speedup vs baseline: 1.0001x; 1.0001x over previous
import functools

import jax
import jax.numpy as jnp
from jax import lax
from jax.experimental import pallas as pl
from jax.experimental.pallas import tpu as pltpu
from jax.experimental.pallas import tpu_sc as plsc

VOCAB = 32000
PAD = 0
SMOOTH = 0.1
CONF = 1.0 - SMOOTH
LOW = SMOOTH / (VOCAB - 2)

ROWS = 256
CHUNK = 6400
NCHUNK = VOCAB // CHUNK

N_TOKENS = 8192
SC_CORES = 2
SC_SUBCORES = 16
SC_WORKERS = SC_CORES * SC_SUBCORES
PER_TILE = N_TOKENS // SC_WORKERS
LANES = 16
IDX_W = 128
IDX_ROWS = PER_TILE // IDX_W
SUBCOLS = 128
SUBROWS_PER_TOKEN = VOCAB // SUBCOLS


@functools.partial(
    pl.kernel,
    mesh=plsc.VectorSubcoreMesh(core_axis_name="c", subcore_axis_name="s"),
    out_type=jax.ShapeDtypeStruct((N_TOKENS, SUBCOLS), jnp.float32),
    scratch_types=[
        pltpu.VMEM((PER_TILE,), jnp.int32),
        pltpu.VMEM((IDX_ROWS, IDX_W), jnp.int32),
        pltpu.VMEM((IDX_W, SUBCOLS), jnp.float32),
        pltpu.VMEM((IDX_W, SUBCOLS), jnp.float32),
        pltpu.SemaphoreType.DMA,
    ],
)
def _sc_gather_pt(tgt_hbm, pred_hbm, out_hbm, tgt_v, idx_v, rows_a, rows_b,
                  sem):
    wid = lax.axis_index("s") * SC_CORES + lax.axis_index("c")
    base = wid * PER_TILE
    pltpu.sync_copy(tgt_hbm.at[pl.ds(base, PER_TILE)], tgt_v)
    for k in range(PER_TILE // LANES):
        q, off = divmod(k * LANES, IDX_W)
        toks = (base + k * LANES) + lax.iota(jnp.int32, LANES)
        t16 = tgt_v[pl.ds(k * LANES, LANES)]
        idx_v[q, pl.ds(off, LANES)] = (
            toks * SUBROWS_PER_TOKEN + jnp.right_shift(t16, 7))
    rows_bufs = (rows_a, rows_b)
    cps = [pltpu.async_copy(pred_hbm.at[idx_v.at[q]], rows_bufs[q], sem)
           for q in range(IDX_ROWS)]
    for q in range(IDX_ROWS):
        cps[q].wait()
        pltpu.sync_copy(rows_bufs[q],
                        out_hbm.at[pl.ds(base + q * IDX_W, IDX_W)])


def _tc_body(pred_ref, tgt_ref, pt_in_ref, out_ref, m_ref, se_ref, sp_ref,
             p0_ref, num_ref, den_ref):
    i = pl.program_id(0)
    j = pl.program_id(1)

    chunk = pred_ref[...]

    @pl.when(j == 0)
    def _init_row_block():
        m_ref[...] = jnp.full((ROWS, 1), -jnp.inf, jnp.float32)
        se_ref[...] = jnp.zeros((ROWS, 1), jnp.float32)
        sp_ref[...] = jnp.zeros((ROWS, 1), jnp.float32)
        p0_ref[...] = chunk[:, 0:1]

    @pl.when((i == 0) & (j == 0))
    def _init_accum():
        num_ref[0, 0] = 0.0
        den_ref[0, 0] = 0.0

    m_old = m_ref[...]
    m_new = jnp.maximum(m_old, jnp.max(chunk, axis=1, keepdims=True))
    alpha = jnp.exp(m_old - m_new)
    cse = jnp.sum(jnp.exp(chunk - m_new), axis=1, keepdims=True)
    se_ref[...] = se_ref[...] * alpha + cse
    sp_ref[...] = sp_ref[...] + jnp.sum(chunk, axis=1, keepdims=True)
    m_ref[...] = m_new

    @pl.when(j == NCHUNK - 1)
    def _finish_row_block():
        tgt = tgt_ref[...]
        sub = pt_in_ref[...]
        tcol = jnp.bitwise_and(tgt, SUBCOLS - 1)
        lanes = jax.lax.broadcasted_iota(jnp.int32, (ROWS, SUBCOLS), 1)
        pt = jnp.sum(jnp.where(lanes == tcol, sub, 0.0), axis=1,
                     keepdims=True)
        L = m_ref[...] + jnp.log(se_ref[...])
        loss = (L - LOW * (sp_ref[...] - p0_ref[...] - pt) - CONF * pt)
        maskf = (tgt != PAD).astype(jnp.float32)
        num_ref[0, 0] += jnp.sum(loss * maskf)
        den_ref[0, 0] += jnp.sum(maskf)

    @pl.when((i == pl.num_programs(0) - 1) & (j == NCHUNK - 1))
    def _emit():
        out_ref[...] = jnp.full(
            (1, 1), num_ref[0, 0] / jnp.maximum(den_ref[0, 0], 1.0),
            jnp.float32)


def kernel(pred, target):
    n = pred.shape[0]
    sub = _sc_gather_pt(target, pred.reshape(-1, SUBCOLS))
    tgt2d = target.reshape(n, 1)
    out = pl.pallas_call(
        _tc_body,
        grid=(n // ROWS, NCHUNK),
        in_specs=[
            pl.BlockSpec((ROWS, CHUNK), lambda i, j: (i, j)),
            pl.BlockSpec((ROWS, 1), lambda i, j: (i, 0)),
            pl.BlockSpec((ROWS, SUBCOLS), lambda i, j: (i, 0)),
        ],
        out_specs=pl.BlockSpec((1, 1), lambda i, j: (0, 0)),
        out_shape=jax.ShapeDtypeStruct((1, 1), jnp.float32),
        scratch_shapes=[
            pltpu.VMEM((ROWS, 1), jnp.float32),
            pltpu.VMEM((ROWS, 1), jnp.float32),
            pltpu.VMEM((ROWS, 1), jnp.float32),
            pltpu.VMEM((ROWS, 1), jnp.float32),
            pltpu.SMEM((1, 1), jnp.float32),
            pltpu.SMEM((1, 1), jnp.float32),
        ],
    )(pred, tgt2d, sub)
    return out[0, 0]


# speedup vs baseline: 2.5965x; 2.5963x over previous
import functools

import jax
import jax.numpy as jnp
from jax import lax
from jax.experimental import pallas as pl
from jax.experimental.pallas import tpu as pltpu
from jax.experimental.pallas import tpu_sc as plsc

VOCAB = 32000
PAD = 0
SMOOTH = 0.1
CONF = 1.0 - SMOOTH
LOW = SMOOTH / (VOCAB - 2)

ROWS = 256
CHUNK = 6400
NCHUNK = VOCAB // CHUNK

N_TOKENS = 8192
SC_CORES = 2
SC_SUBCORES = 16
SC_WORKERS = SC_CORES * SC_SUBCORES
PER_TILE = N_TOKENS // SC_WORKERS
LANES = 16
IDX_W = 128
IDX_ROWS = PER_TILE // IDX_W
SUBCOLS = 128
SUBROWS_PER_TOKEN = VOCAB // SUBCOLS


@functools.partial(
    pl.kernel,
    mesh=plsc.VectorSubcoreMesh(core_axis_name="c", subcore_axis_name="s"),
    out_type=jax.ShapeDtypeStruct((N_TOKENS, SUBCOLS), jnp.float32),
    scratch_types=[
        pltpu.VMEM((PER_TILE,), jnp.int32),
        pltpu.VMEM((IDX_ROWS, IDX_W), jnp.int32),
        pltpu.VMEM((IDX_W, SUBCOLS), jnp.float32),
        pltpu.VMEM((IDX_W, SUBCOLS), jnp.float32),
        pltpu.SemaphoreType.DMA,
    ],
)
def _sc_gather_pt(tgt_hbm, pred_hbm, out_hbm, tgt_v, idx_v, rows_a, rows_b,
                  sem):
    wid = lax.axis_index("s") * SC_CORES + lax.axis_index("c")
    base = wid * PER_TILE
    pltpu.sync_copy(tgt_hbm.at[pl.ds(base, PER_TILE)], tgt_v)
    for k in range(PER_TILE // LANES):
        q, off = divmod(k * LANES, IDX_W)
        toks = (base + k * LANES) + lax.iota(jnp.int32, LANES)
        t16 = tgt_v[pl.ds(k * LANES, LANES)]
        idx_v[q, pl.ds(off, LANES)] = (
            (jnp.right_shift(toks, 3) * SUBROWS_PER_TOKEN
             + jnp.right_shift(t16, 7)) * 8 + jnp.bitwise_and(toks, 7))
    rows_bufs = (rows_a, rows_b)
    cps = [pltpu.async_copy(pred_hbm.at[idx_v.at[q]], rows_bufs[q], sem)
           for q in range(IDX_ROWS)]
    for q in range(IDX_ROWS):
        cps[q].wait()
        pltpu.sync_copy(rows_bufs[q],
                        out_hbm.at[pl.ds(base + q * IDX_W, IDX_W)])


def _tc_body(pred_ref, tgt_ref, pt_in_ref, out_ref, m_ref, se_ref, sp_ref,
             p0_ref, num_ref, den_ref):
    i = pl.program_id(0)
    j = pl.program_id(1)

    chunk = pred_ref[...]

    @pl.when(j == 0)
    def _init_row_block():
        m_ref[...] = jnp.full((ROWS, 1), -jnp.inf, jnp.float32)
        se_ref[...] = jnp.zeros((ROWS, 1), jnp.float32)
        sp_ref[...] = jnp.zeros((ROWS, 1), jnp.float32)
        p0_ref[...] = chunk[:, 0:1]

    @pl.when((i == 0) & (j == 0))
    def _init_accum():
        num_ref[0, 0] = 0.0
        den_ref[0, 0] = 0.0

    m_old = m_ref[...]
    m_new = jnp.maximum(m_old, jnp.max(chunk, axis=1, keepdims=True))
    alpha = jnp.exp(m_old - m_new)
    cse = jnp.sum(jnp.exp(chunk - m_new), axis=1, keepdims=True)
    se_ref[...] = se_ref[...] * alpha + cse
    sp_ref[...] = sp_ref[...] + jnp.sum(chunk, axis=1, keepdims=True)
    m_ref[...] = m_new

    @pl.when(j == NCHUNK - 1)
    def _finish_row_block():
        tgt = tgt_ref[...]
        sub = pt_in_ref[...]
        tcol = jnp.bitwise_and(tgt, SUBCOLS - 1)
        lanes = jax.lax.broadcasted_iota(jnp.int32, (ROWS, SUBCOLS), 1)
        pt = jnp.sum(jnp.where(lanes == tcol, sub, 0.0), axis=1,
                     keepdims=True)
        L = m_ref[...] + jnp.log(se_ref[...])
        loss = (L - LOW * (sp_ref[...] - p0_ref[...] - pt) - CONF * pt)
        maskf = (tgt != PAD).astype(jnp.float32)
        num_ref[0, 0] += jnp.sum(loss * maskf)
        den_ref[0, 0] += jnp.sum(maskf)

    @pl.when((i == pl.num_programs(0) - 1) & (j == NCHUNK - 1))
    def _emit():
        out_ref[...] = jnp.full(
            (1, 1), num_ref[0, 0] / jnp.maximum(den_ref[0, 0], 1.0),
            jnp.float32)


def kernel(pred, target):
    n = pred.shape[0]
    tiled_view = pred.reshape(n // 8, 8, SUBROWS_PER_TOKEN, SUBCOLS)
    tiled_view = tiled_view.transpose(0, 2, 1, 3).reshape(-1, SUBCOLS)
    sub = _sc_gather_pt(target, tiled_view)
    tgt2d = target.reshape(n, 1)
    out = pl.pallas_call(
        _tc_body,
        grid=(n // ROWS, NCHUNK),
        in_specs=[
            pl.BlockSpec((ROWS, CHUNK), lambda i, j: (i, j)),
            pl.BlockSpec((ROWS, 1), lambda i, j: (i, 0)),
            pl.BlockSpec((ROWS, SUBCOLS), lambda i, j: (i, 0)),
        ],
        out_specs=pl.BlockSpec((1, 1), lambda i, j: (0, 0)),
        out_shape=jax.ShapeDtypeStruct((1, 1), jnp.float32),
        scratch_shapes=[
            pltpu.VMEM((ROWS, 1), jnp.float32),
            pltpu.VMEM((ROWS, 1), jnp.float32),
            pltpu.VMEM((ROWS, 1), jnp.float32),
            pltpu.VMEM((ROWS, 1), jnp.float32),
            pltpu.SMEM((1, 1), jnp.float32),
            pltpu.SMEM((1, 1), jnp.float32),
        ],
    )(pred, tgt2d, sub)
    return out[0, 0]


# speedup vs baseline: 2.8246x; 1.0878x over previous
import functools

import jax
import jax.numpy as jnp
from jax import lax
from jax.experimental import pallas as pl
from jax.experimental.pallas import tpu as pltpu
from jax.experimental.pallas import tpu_sc as plsc

VOCAB = 32000
PAD = 0
SMOOTH = 0.1
CONF = 1.0 - SMOOTH
LOW = SMOOTH / (VOCAB - 2)

ROWS = 512
CHUNK = 6400
NCHUNK = VOCAB // CHUNK

N_TOKENS = 8192
SC_CORES = 2
SC_SUBCORES = 16
SC_WORKERS = SC_CORES * SC_SUBCORES
PER_TILE = N_TOKENS // SC_WORKERS
LANES = 16
IDX_W = 128
IDX_ROWS = PER_TILE // IDX_W
SUBCOLS = 128
SUBROWS_PER_TOKEN = VOCAB // SUBCOLS


@functools.partial(
    pl.kernel,
    mesh=plsc.VectorSubcoreMesh(core_axis_name="c", subcore_axis_name="s"),
    out_type=jax.ShapeDtypeStruct((N_TOKENS, SUBCOLS), jnp.float32),
    scratch_types=[
        pltpu.VMEM((PER_TILE,), jnp.int32),
        pltpu.VMEM((IDX_ROWS, IDX_W), jnp.int32),
        pltpu.VMEM((IDX_W, SUBCOLS), jnp.float32),
        pltpu.VMEM((IDX_W, SUBCOLS), jnp.float32),
        pltpu.SemaphoreType.DMA,
    ],
)
def _sc_gather_pt(tgt_hbm, pred_hbm, out_hbm, tgt_v, idx_v, rows_a, rows_b,
                  sem):
    wid = lax.axis_index("s") * SC_CORES + lax.axis_index("c")
    base = wid * PER_TILE
    pltpu.sync_copy(tgt_hbm.at[pl.ds(base, PER_TILE)], tgt_v)
    for k in range(PER_TILE // LANES):
        q, off = divmod(k * LANES, IDX_W)
        toks = (base + k * LANES) + lax.iota(jnp.int32, LANES)
        t16 = tgt_v[pl.ds(k * LANES, LANES)]
        idx_v[q, pl.ds(off, LANES)] = (
            (jnp.right_shift(toks, 3) * SUBROWS_PER_TOKEN
             + jnp.right_shift(t16, 7)) * 8 + jnp.bitwise_and(toks, 7))
    rows_bufs = (rows_a, rows_b)
    cps = [pltpu.async_copy(pred_hbm.at[idx_v.at[q]], rows_bufs[q], sem)
           for q in range(IDX_ROWS)]
    for q in range(IDX_ROWS):
        cps[q].wait()
        pltpu.sync_copy(rows_bufs[q],
                        out_hbm.at[pl.ds(base + q * IDX_W, IDX_W)])


def _tc_body(pred_ref, tgt_ref, pt_in_ref, out_ref, m_ref, se_ref, sp_ref,
             p0_ref, num_ref, den_ref):
    i = pl.program_id(0)
    j = pl.program_id(1)

    chunk = pred_ref[...]

    @pl.when(j == 0)
    def _init_row_block():
        m_ref[...] = jnp.full((ROWS, 1), -jnp.inf, jnp.float32)
        se_ref[...] = jnp.zeros((ROWS, 1), jnp.float32)
        sp_ref[...] = jnp.zeros((ROWS, 1), jnp.float32)
        p0_ref[...] = chunk[:, 0:1]

    @pl.when((i == 0) & (j == 0))
    def _init_accum():
        num_ref[0, 0] = 0.0
        den_ref[0, 0] = 0.0

    m_old = m_ref[...]
    m_new = jnp.maximum(m_old, jnp.max(chunk, axis=1, keepdims=True))
    alpha = jnp.exp(m_old - m_new)
    cse = jnp.sum(jnp.exp(chunk - m_new), axis=1, keepdims=True)
    se_ref[...] = se_ref[...] * alpha + cse
    sp_ref[...] = sp_ref[...] + jnp.sum(chunk, axis=1, keepdims=True)
    m_ref[...] = m_new

    @pl.when(j == NCHUNK - 1)
    def _finish_row_block():
        tgt = tgt_ref[...]
        sub = pt_in_ref[...]
        tcol = jnp.bitwise_and(tgt, SUBCOLS - 1)
        lanes = jax.lax.broadcasted_iota(jnp.int32, (ROWS, SUBCOLS), 1)
        pt = jnp.sum(jnp.where(lanes == tcol, sub, 0.0), axis=1,
                     keepdims=True)
        L = m_ref[...] + jnp.log(se_ref[...])
        loss = (L - LOW * (sp_ref[...] - p0_ref[...] - pt) - CONF * pt)
        maskf = (tgt != PAD).astype(jnp.float32)
        num_ref[0, 0] += jnp.sum(loss * maskf)
        den_ref[0, 0] += jnp.sum(maskf)

    @pl.when((i == pl.num_programs(0) - 1) & (j == NCHUNK - 1))
    def _emit():
        out_ref[...] = jnp.full(
            (1, 1), num_ref[0, 0] / jnp.maximum(den_ref[0, 0], 1.0),
            jnp.float32)


def kernel(pred, target):
    n = pred.shape[0]
    tiled_view = pred.reshape(n // 8, 8, SUBROWS_PER_TOKEN, SUBCOLS)
    tiled_view = tiled_view.transpose(0, 2, 1, 3).reshape(-1, SUBCOLS)
    sub = _sc_gather_pt(target, tiled_view)
    tgt2d = target.reshape(n, 1)
    out = pl.pallas_call(
        _tc_body,
        grid=(n // ROWS, NCHUNK),
        in_specs=[
            pl.BlockSpec((ROWS, CHUNK), lambda i, j: (i, j)),
            pl.BlockSpec((ROWS, 1), lambda i, j: (i, 0)),
            pl.BlockSpec((ROWS, SUBCOLS), lambda i, j: (i, 0)),
        ],
        out_specs=pl.BlockSpec((1, 1), lambda i, j: (0, 0)),
        out_shape=jax.ShapeDtypeStruct((1, 1), jnp.float32),
        scratch_shapes=[
            pltpu.VMEM((ROWS, 1), jnp.float32),
            pltpu.VMEM((ROWS, 1), jnp.float32),
            pltpu.VMEM((ROWS, 1), jnp.float32),
            pltpu.VMEM((ROWS, 1), jnp.float32),
            pltpu.SMEM((1, 1), jnp.float32),
            pltpu.SMEM((1, 1), jnp.float32),
        ],
    )(pred, tgt2d, sub)
    return out[0, 0]


# speedup vs baseline: 3.0851x; 1.0923x over previous
import functools

import jax
import jax.numpy as jnp
from jax import lax
from jax.experimental import pallas as pl
from jax.experimental.pallas import tpu as pltpu
from jax.experimental.pallas import tpu_sc as plsc

VOCAB = 32000
PAD = 0
SMOOTH = 0.1
CONF = 1.0 - SMOOTH
LOW = SMOOTH / (VOCAB - 2)

ROWS = 512
CHUNK = 6400
NCHUNK = VOCAB // CHUNK
SUB = 1280
NSUB = CHUNK // SUB

N_TOKENS = 8192
SC_CORES = 2
SC_SUBCORES = 16
SC_WORKERS = SC_CORES * SC_SUBCORES
PER_TILE = N_TOKENS // SC_WORKERS
LANES = 16
IDX_W = 128
IDX_ROWS = PER_TILE // IDX_W
SUBCOLS = 128
SUBROWS_PER_TOKEN = VOCAB // SUBCOLS


@functools.partial(
    pl.kernel,
    mesh=plsc.VectorSubcoreMesh(core_axis_name="c", subcore_axis_name="s"),
    out_type=jax.ShapeDtypeStruct((N_TOKENS, SUBCOLS), jnp.float32),
    scratch_types=[
        pltpu.VMEM((PER_TILE,), jnp.int32),
        pltpu.VMEM((IDX_ROWS, IDX_W), jnp.int32),
        pltpu.VMEM((IDX_W, SUBCOLS), jnp.float32),
        pltpu.VMEM((IDX_W, SUBCOLS), jnp.float32),
        pltpu.SemaphoreType.DMA,
    ],
)
def _sc_gather_pt(tgt_hbm, pred_hbm, out_hbm, tgt_v, idx_v, rows_a, rows_b,
                  sem):
    wid = lax.axis_index("s") * SC_CORES + lax.axis_index("c")
    base = wid * PER_TILE
    pltpu.sync_copy(tgt_hbm.at[pl.ds(base, PER_TILE)], tgt_v)
    for k in range(PER_TILE // LANES):
        q, off = divmod(k * LANES, IDX_W)
        toks = (base + k * LANES) + lax.iota(jnp.int32, LANES)
        t16 = tgt_v[pl.ds(k * LANES, LANES)]
        idx_v[q, pl.ds(off, LANES)] = (
            (jnp.right_shift(toks, 3) * SUBROWS_PER_TOKEN
             + jnp.right_shift(t16, 7)) * 8 + jnp.bitwise_and(toks, 7))
    rows_bufs = (rows_a, rows_b)
    cps = [pltpu.async_copy(pred_hbm.at[idx_v.at[q]], rows_bufs[q], sem)
           for q in range(IDX_ROWS)]
    for q in range(IDX_ROWS):
        cps[q].wait()
        pltpu.sync_copy(rows_bufs[q],
                        out_hbm.at[pl.ds(base + q * IDX_W, IDX_W)])


def _tc_body(pred_ref, tgt_ref, pt_in_ref, out_ref, m_ref, se_ref, sp_ref,
             p0_ref, num_ref, den_ref):
    i = pl.program_id(0)
    j = pl.program_id(1)

    @pl.when(j == 0)
    def _init_row_block():
        m_ref[...] = jnp.full((ROWS, 1), -jnp.inf, jnp.float32)
        se_ref[...] = jnp.zeros((ROWS, 1), jnp.float32)
        sp_ref[...] = jnp.zeros((ROWS, 1), jnp.float32)
        p0_ref[...] = pred_ref[:, 0:1]

    @pl.when((i == 0) & (j == 0))
    def _init_accum():
        num_ref[0, 0] = 0.0
        den_ref[0, 0] = 0.0

    m = m_ref[...]
    se = se_ref[...]
    sp = sp_ref[...]
    for s in range(NSUB):
        c = pred_ref[:, pl.ds(s * SUB, SUB)]
        m_new = jnp.maximum(m, jnp.max(c, axis=1, keepdims=True))
        alpha = jnp.exp(m - m_new)
        cse = jnp.sum(jnp.exp(c - m_new), axis=1, keepdims=True)
        se = se * alpha + cse
        sp = sp + jnp.sum(c, axis=1, keepdims=True)
        m = m_new
    m_ref[...] = m
    se_ref[...] = se
    sp_ref[...] = sp

    @pl.when(j == NCHUNK - 1)
    def _finish_row_block():
        tgt = tgt_ref[...]
        sub = pt_in_ref[...]
        tcol = jnp.bitwise_and(tgt, SUBCOLS - 1)
        lanes = jax.lax.broadcasted_iota(jnp.int32, (ROWS, SUBCOLS), 1)
        pt = jnp.sum(jnp.where(lanes == tcol, sub, 0.0), axis=1,
                     keepdims=True)
        L = m_ref[...] + jnp.log(se_ref[...])
        loss = (L - LOW * (sp_ref[...] - p0_ref[...] - pt) - CONF * pt)
        maskf = (tgt != PAD).astype(jnp.float32)
        num_ref[0, 0] += jnp.sum(loss * maskf)
        den_ref[0, 0] += jnp.sum(maskf)

    @pl.when((i == pl.num_programs(0) - 1) & (j == NCHUNK - 1))
    def _emit():
        out_ref[...] = jnp.full(
            (1, 1), num_ref[0, 0] / jnp.maximum(den_ref[0, 0], 1.0),
            jnp.float32)


def kernel(pred, target):
    n = pred.shape[0]
    tiled_view = pred.reshape(n // 8, 8, SUBROWS_PER_TOKEN, SUBCOLS)
    tiled_view = tiled_view.transpose(0, 2, 1, 3).reshape(-1, SUBCOLS)
    sub = _sc_gather_pt(target, tiled_view)
    tgt2d = target.reshape(n, 1)
    out = pl.pallas_call(
        _tc_body,
        grid=(n // ROWS, NCHUNK),
        in_specs=[
            pl.BlockSpec((ROWS, CHUNK), lambda i, j: (i, j)),
            pl.BlockSpec((ROWS, 1), lambda i, j: (i, 0)),
            pl.BlockSpec((ROWS, SUBCOLS), lambda i, j: (i, 0)),
        ],
        out_specs=pl.BlockSpec((1, 1), lambda i, j: (0, 0)),
        out_shape=jax.ShapeDtypeStruct((1, 1), jnp.float32),
        scratch_shapes=[
            pltpu.VMEM((ROWS, 1), jnp.float32),
            pltpu.VMEM((ROWS, 1), jnp.float32),
            pltpu.VMEM((ROWS, 1), jnp.float32),
            pltpu.VMEM((ROWS, 1), jnp.float32),
            pltpu.SMEM((1, 1), jnp.float32),
            pltpu.SMEM((1, 1), jnp.float32),
        ],
    )(pred, tgt2d, sub)
    return out[0, 0]
